# baseline (speedup 1.0000x reference)
import functools

import jax
import jax.numpy as jnp
from jax import lax
from jax.experimental import pallas as pl
from jax.experimental.pallas import tpu as pltpu
from jax.experimental.pallas import tpu_sc as plsc

_NC = 2
_NS = 16
_NW = _NC * _NS
_R = 4


@functools.partial(jax.jit, static_argnames=("b", "s", "d"))
def _emb_lookup(xf, feature_matrix, b, s, d):
    b_per_w = b // _NW
    n_sup = b_per_w // _R
    assert b_per_w % _R == 0 and n_sup % 2 == 0
    splits = []
    off = 0
    while off < s:
        w = min(128, s - off)
        splits.append((off, w))
        off += w
    mesh = plsc.VectorSubcoreMesh(core_axis_name="c", subcore_axis_name="s")

    @functools.partial(
        pl.kernel,
        mesh=mesh,
        compiler_params=pltpu.CompilerParams(use_tc_tiling_on_sc=False),
        out_type=jax.ShapeDtypeStruct((b, s, d), jnp.float32),
        scratch_types=[
            pltpu.VMEM((b_per_w, s), jnp.int32),
            pltpu.VMEM((2, _R, s, d), jnp.float32),
            pltpu.SemaphoreType.DMA,
            pltpu.SemaphoreType.DMA,
            pltpu.SemaphoreType.DMA,
        ],
    )
    def emb(x_hbm, tab_hbm, out_hbm, idx_v, rows_v, gsem, osem0, osem1):
        wid = lax.axis_index("s") * _NC + lax.axis_index("c")
        base_b = wid * b_per_w
        pltpu.sync_copy(x_hbm.at[wid], idx_v)

        def half(t, sup, slot, osem):
            for r in range(_R):
                for off, w in splits:
                    pltpu.async_copy(
                        tab_hbm.at[idx_v.at[sup * _R + r, pl.ds(off, w)]],
                        rows_v.at[slot, r, pl.ds(off, w)],
                        gsem,
                    )
            for r in range(_R):
                for off, w in splits:
                    pltpu.make_async_copy(
                        tab_hbm.at[idx_v.at[r, pl.ds(off, w)]],
                        rows_v.at[slot, r, pl.ds(off, w)],
                        gsem,
                    ).wait()

            @pl.when(t >= 1)
            def _():
                pltpu.make_async_copy(
                    rows_v.at[slot], out_hbm.at[pl.ds(base_b, _R)], osem
                ).wait()

            pltpu.async_copy(
                rows_v.at[slot],
                out_hbm.at[pl.ds(base_b + sup * _R, _R)],
                osem,
            )

        def body(t, carry):
            half(t, 2 * t, 0, osem0)
            half(t, 2 * t + 1, 1, osem1)
            return carry

        lax.fori_loop(0, n_sup // 2, body, 0)
        pltpu.make_async_copy(
            rows_v.at[0], out_hbm.at[pl.ds(base_b, _R)], osem0
        ).wait()
        pltpu.make_async_copy(
            rows_v.at[1], out_hbm.at[pl.ds(base_b, _R)], osem1
        ).wait()

    return emb(xf, feature_matrix)


def kernel(x, feature_matrix):
    b, s = x.shape
    v, d = feature_matrix.shape
    assert b % _NW == 0
    xf = x.reshape(_NW, b // _NW, s).astype(jnp.int32)
    return _emb_lookup(xf, feature_matrix, b, s, d)


# speedup vs baseline: 2.2312x; 2.2312x over previous
import functools

import jax
import jax.numpy as jnp
from jax import lax
from jax.experimental import pallas as pl
from jax.experimental.pallas import tpu as pltpu
from jax.experimental.pallas import tpu_sc as plsc

_NC = 2
_NS = 16
_NW = _NC * _NS
_L = 16
_BT = 128


@functools.partial(jax.jit, static_argnames=("b", "s", "d"))
def _emb_lookup(xt, feature_matrix, b, s, d):
    n_bt = b // _BT
    ft = d // 8
    assert n_bt == _NW and d % 8 == 0
    mesh = plsc.VectorSubcoreMesh(core_axis_name="c", subcore_axis_name="s")

    @functools.partial(
        pl.kernel,
        mesh=mesh,
        compiler_params=pltpu.CompilerParams(
            use_tc_tiling_on_sc=False, needs_layout_passes=False
        ),
        out_type=jax.ShapeDtypeStruct((s, ft, n_bt, 8, _BT), jnp.float32),
        scratch_types=[
            pltpu.VMEM((s, _BT), jnp.int32),
            pltpu.VMEM((2, _BT, d), jnp.float32),
            pltpu.VMEM((2, ft, 8, _BT), jnp.float32),
            pltpu.SemaphoreType.DMA,
            pltpu.SemaphoreType.DMA,
            pltpu.SemaphoreType.DMA,
            pltpu.SemaphoreType.DMA,
        ],
    )
    def emb(x_hbm, tab_hbm, out_hbm, idx_v, rows_v, tile_v,
            gsem0, gsem1, osem0, osem1):
        wid = lax.axis_index("s") * _NC + lax.axis_index("c")
        pltpu.sync_copy(x_hbm.at[wid], idx_v)

        iota = lax.iota(jnp.int32, _L)
        gsems = (gsem0, gsem1)
        osems = (osem0, osem1)

        pltpu.async_copy(tab_hbm.at[idx_v.at[0]], rows_v.at[0], gsem0)

        def unit(u, slot):
            pltpu.make_async_copy(
                tab_hbm.at[idx_v.at[0]], rows_v.at[slot], gsems[slot]
            ).wait()

            @pl.when(u + 1 < s)
            def _():
                pltpu.async_copy(
                    tab_hbm.at[idx_v.at[u + 1]],
                    rows_v.at[1 - slot],
                    gsems[1 - slot],
                )

            @pl.when(u >= 2)
            def _():
                pltpu.make_async_copy(
                    tile_v.at[slot],
                    out_hbm.at[0, pl.ds(0, ft), wid],
                    osems[slot],
                ).wait()

            rb = rows_v.at[slot]
            for f in range(d):
                fc = jnp.full((_L,), f, jnp.int32)
                for g in range(_BT // _L):
                    vals = plsc.load_gather(rb, [iota + (g * _L), fc])
                    tile_v[slot, f // 8, f % 8, pl.ds(g * _L, _L)] = vals

            pltpu.async_copy(
                tile_v.at[slot],
                out_hbm.at[u, pl.ds(0, ft), wid],
                osems[slot],
            )

        def body(t, carry):
            unit(2 * t, 0)
            unit(2 * t + 1, 1)
            return carry

        lax.fori_loop(0, s // 2, body, 0)
        for slot in range(2):
            pltpu.make_async_copy(
                tile_v.at[slot],
                out_hbm.at[0, pl.ds(0, ft), wid],
                osems[slot],
            ).wait()

    return emb(xt, feature_matrix)


def kernel(x, feature_matrix):
    b, s = x.shape
    v, d = feature_matrix.shape
    assert b % _BT == 0 and b // _BT == _NW and s % 2 == 0
    xt = x.astype(jnp.int32).T.reshape(s, _NW, _BT).transpose(1, 0, 2)
    out5 = _emb_lookup(xt, feature_matrix, b, s, d)
    return out5.transpose(2, 4, 0, 1, 3).reshape(b, s, d)


# speedup vs baseline: 2.5815x; 1.1570x over previous
import functools

import jax
import jax.numpy as jnp
from jax import lax
from jax.experimental import pallas as pl
from jax.experimental.pallas import tpu as pltpu
from jax.experimental.pallas import tpu_sc as plsc

_NC = 2
_NS = 16
_NW = _NC * _NS
_L = 16
_BT = 128


@functools.partial(jax.jit, static_argnames=("b", "s", "d"))
def _emb_lookup(xt, feature_matrix, b, s, d):
    n_bt = b // _BT
    ft = d // 8
    assert n_bt == _NW and d % 8 == 0
    mesh = plsc.VectorSubcoreMesh(core_axis_name="c", subcore_axis_name="s")

    @functools.partial(
        pl.kernel,
        mesh=mesh,
        compiler_params=pltpu.CompilerParams(
            use_tc_tiling_on_sc=False, needs_layout_passes=False
        ),
        out_type=jax.ShapeDtypeStruct((s, ft, n_bt, 8, _BT), jnp.float32),
        scratch_types=[
            pltpu.VMEM((s, _BT), jnp.int32),
            pltpu.VMEM((2, _BT, d), jnp.float32),
            pltpu.VMEM((2, ft, 8, _BT), jnp.float32),
            pltpu.SemaphoreType.DMA,
            pltpu.SemaphoreType.DMA,
            pltpu.SemaphoreType.DMA,
            pltpu.SemaphoreType.DMA,
        ],
    )
    def emb(x_hbm, tab_hbm, out_hbm, idx_v, rows_v, tile_v,
            gsem0, gsem1, osem0, osem1):
        wid = lax.axis_index("s") * _NC + lax.axis_index("c")
        pltpu.sync_copy(x_hbm.at[wid], idx_v)

        iota = lax.iota(jnp.int32, _L)
        bi_vecs = [iota + (g * _L) for g in range(_BT // _L)]
        f_vecs = [jnp.full((_L,), f, jnp.int32) for f in range(d)]
        gsems = (gsem0, gsem1)
        osems = (osem0, osem1)

        pltpu.async_copy(tab_hbm.at[idx_v.at[0]], rows_v.at[0], gsem0)

        def unit(u, slot):
            pltpu.make_async_copy(
                tab_hbm.at[idx_v.at[0]], rows_v.at[slot], gsems[slot]
            ).wait()

            @pl.when(u + 1 < s)
            def _():
                pltpu.async_copy(
                    tab_hbm.at[idx_v.at[u + 1]],
                    rows_v.at[1 - slot],
                    gsems[1 - slot],
                )

            @pl.when(u >= 2)
            def _():
                pltpu.make_async_copy(
                    tile_v.at[slot],
                    out_hbm.at[0, pl.ds(0, ft), wid],
                    osems[slot],
                ).wait()

            rb = rows_v.at[slot]
            pairs = [(f, g) for f in range(d) for g in range(_BT // _L)]
            lag = 8
            pend = {}
            for i, (f, g) in enumerate(pairs):
                pend[i] = plsc.load_gather(rb, [bi_vecs[g], f_vecs[f]])
                if i >= lag:
                    pf, pg = pairs[i - lag]
                    tile_v[slot, pf // 8, pf % 8, pl.ds(pg * _L, _L)] = (
                        pend.pop(i - lag)
                    )
            for i in range(len(pairs) - lag, len(pairs)):
                pf, pg = pairs[i]
                tile_v[slot, pf // 8, pf % 8, pl.ds(pg * _L, _L)] = (
                    pend.pop(i)
                )

            pltpu.async_copy(
                tile_v.at[slot],
                out_hbm.at[u, pl.ds(0, ft), wid],
                osems[slot],
            )

        def body(t, carry):
            unit(2 * t, 0)
            unit(2 * t + 1, 1)
            return carry

        lax.fori_loop(0, s // 2, body, 0)
        for slot in range(2):
            pltpu.make_async_copy(
                tile_v.at[slot],
                out_hbm.at[0, pl.ds(0, ft), wid],
                osems[slot],
            ).wait()

    return emb(xt, feature_matrix)


def kernel(x, feature_matrix):
    b, s = x.shape
    v, d = feature_matrix.shape
    assert b % _BT == 0 and b // _BT == _NW and s % 2 == 0
    xt = x.astype(jnp.int32).T.reshape(s, _NW, _BT).transpose(1, 0, 2)
    out5 = _emb_lookup(xt, feature_matrix, b, s, d)
    return out5.transpose(2, 4, 0, 1, 3).reshape(b, s, d)


# speedup vs baseline: 9.7141x; 3.7630x over previous
import functools

import jax
import jax.numpy as jnp
from jax import lax
from jax.experimental import pallas as pl
from jax.experimental.pallas import tpu as pltpu
from jax.experimental.pallas import tpu_sc as plsc

_NC = 2
_NS = 16
_NW = _NC * _NS
_L = 16
_BT = 128


@functools.partial(jax.jit, static_argnames=("b", "s", "d"))
def _emb_lookup(xt, feature_matrix, b, s, d):
    n_bt = b // _BT
    ft = d // 8
    assert n_bt == _NW and d % 8 == 0
    v = feature_matrix.shape[0]
    mesh = plsc.VectorSubcoreMesh(core_axis_name="c", subcore_axis_name="s")

    @functools.partial(
        pl.kernel,
        mesh=mesh,
        compiler_params=pltpu.CompilerParams(
            use_tc_tiling_on_sc=False, needs_layout_passes=False
        ),
        out_type=jax.ShapeDtypeStruct((s, ft, n_bt, 8, _BT), jnp.float32),
        scratch_types=[
            pltpu.VMEM((v, d), jnp.float32),
            pltpu.VMEM((s, _BT), jnp.int32),
            pltpu.VMEM((2, ft, 8, _BT), jnp.float32),
            pltpu.SemaphoreType.DMA,
            pltpu.SemaphoreType.DMA,
        ],
    )
    def emb(x_hbm, tab_hbm, out_hbm, tab_v, idx_v, tile_v, osem0, osem1):
        wid = lax.axis_index("s") * _NC + lax.axis_index("c")
        pltpu.sync_copy(tab_hbm, tab_v)
        pltpu.sync_copy(x_hbm.at[wid], idx_v)

        f_vecs = [jnp.full((_L,), f, jnp.int32) for f in range(d)]
        osems = (osem0, osem1)

        def unit(u, slot):
            @pl.when(u >= 2)
            def _():
                pltpu.make_async_copy(
                    tile_v.at[slot],
                    out_hbm.at[0, pl.ds(0, ft), wid],
                    osems[slot],
                ).wait()

            idx_gs = [
                idx_v[u, pl.ds(g * _L, _L)] for g in range(_BT // _L)
            ]
            pairs = [(f, g) for g in range(_BT // _L) for f in range(d)]
            lag = 8
            pend = {}
            for i, (f, g) in enumerate(pairs):
                pend[i] = plsc.load_gather(tab_v, [idx_gs[g], f_vecs[f]])
                if i >= lag:
                    pf, pg = pairs[i - lag]
                    tile_v[slot, pf // 8, pf % 8, pl.ds(pg * _L, _L)] = (
                        pend.pop(i - lag)
                    )
            for i in range(len(pairs) - lag, len(pairs)):
                pf, pg = pairs[i]
                tile_v[slot, pf // 8, pf % 8, pl.ds(pg * _L, _L)] = (
                    pend.pop(i)
                )

            pltpu.async_copy(
                tile_v.at[slot],
                out_hbm.at[u, pl.ds(0, ft), wid],
                osems[slot],
            )

        def body(t, carry):
            unit(2 * t, 0)
            unit(2 * t + 1, 1)
            return carry

        lax.fori_loop(0, s // 2, body, 0)
        for slot in range(2):
            pltpu.make_async_copy(
                tile_v.at[slot],
                out_hbm.at[0, pl.ds(0, ft), wid],
                osems[slot],
            ).wait()

    return emb(xt, feature_matrix)


def kernel(x, feature_matrix):
    b, s = x.shape
    v, d = feature_matrix.shape
    assert b % _BT == 0 and b // _BT == _NW and s % 2 == 0
    xt = x.astype(jnp.int32).T.reshape(s, _NW, _BT).transpose(1, 0, 2)
    out5 = _emb_lookup(xt, feature_matrix, b, s, d)
    return out5.transpose(2, 4, 0, 1, 3).reshape(b, s, d)


# speedup vs baseline: 9.9924x; 1.0286x over previous
import functools

import jax
import jax.numpy as jnp
from jax import lax
from jax.experimental import pallas as pl
from jax.experimental.pallas import tpu as pltpu
from jax.experimental.pallas import tpu_sc as plsc

_NC = 2
_NS = 16
_NW = _NC * _NS
_L = 16
_BT = 128


@functools.partial(jax.jit, static_argnames=("b", "s", "d"))
def _emb_lookup(xt, feature_matrix, b, s, d):
    n_bt = b // _BT
    ft = d // 8
    st_n = s // 8
    assert n_bt == _NW and d % 8 == 0
    v = feature_matrix.shape[0]
    mesh = plsc.VectorSubcoreMesh(core_axis_name="c", subcore_axis_name="s")

    @functools.partial(
        pl.kernel,
        mesh=mesh,
        compiler_params=pltpu.CompilerParams(
            use_tc_tiling_on_sc=False, needs_layout_passes=False
        ),
        out_type=jax.ShapeDtypeStruct((s, ft, n_bt, 8, _BT), jnp.float32),
        scratch_types=[
            pltpu.VMEM((v, d), jnp.float32),
            pltpu.VMEM((st_n, 8, _BT), jnp.int32),
            pltpu.VMEM((2, ft, 8, _BT), jnp.float32),
            pltpu.SemaphoreType.DMA,
            pltpu.SemaphoreType.DMA,
        ],
    )
    def emb(x_hbm, tab_hbm, out_hbm, tab_v, idx_v, tile_v, osem0, osem1):
        wid = lax.axis_index("s") * _NC + lax.axis_index("c")
        pltpu.sync_copy(tab_hbm, tab_v)
        pltpu.sync_copy(x_hbm.at[:, wid], idx_v)

        f_vecs = [jnp.full((_L,), f, jnp.int32) for f in range(d)]
        osems = (osem0, osem1)

        def unit(u, slot):
            @pl.when(u >= 2)
            def _():
                pltpu.make_async_copy(
                    tile_v.at[slot],
                    out_hbm.at[0, pl.ds(0, ft), wid],
                    osems[slot],
                ).wait()

            st = u // 8
            si = lax.rem(u, 8)
            idx_gs = [
                idx_v[st, si, pl.ds(g * _L, _L)] for g in range(_BT // _L)
            ]
            pairs = [(f, g) for g in range(_BT // _L) for f in range(d)]
            lag = 8
            pend = {}
            for i, (f, g) in enumerate(pairs):
                pend[i] = plsc.load_gather(tab_v, [idx_gs[g], f_vecs[f]])
                if i >= lag:
                    pf, pg = pairs[i - lag]
                    tile_v[slot, pf // 8, pf % 8, pl.ds(pg * _L, _L)] = (
                        pend.pop(i - lag)
                    )
            for i in range(len(pairs) - lag, len(pairs)):
                pf, pg = pairs[i]
                tile_v[slot, pf // 8, pf % 8, pl.ds(pg * _L, _L)] = (
                    pend.pop(i)
                )

            pltpu.async_copy(
                tile_v.at[slot],
                out_hbm.at[u, pl.ds(0, ft), wid],
                osems[slot],
            )

        def body(t, carry):
            unit(2 * t, 0)
            unit(2 * t + 1, 1)
            return carry

        lax.fori_loop(0, s // 2, body, 0)
        for slot in range(2):
            pltpu.make_async_copy(
                tile_v.at[slot],
                out_hbm.at[0, pl.ds(0, ft), wid],
                osems[slot],
            ).wait()

    return emb(xt, feature_matrix)


def kernel(x, feature_matrix):
    b, s = x.shape
    v, d = feature_matrix.shape
    assert b % _BT == 0 and b // _BT == _NW and s % 8 == 0
    xt = (
        x.astype(jnp.int32)
        .T.reshape(s // 8, 8, _NW, _BT)
        .transpose(0, 2, 1, 3)
    )
    out5 = _emb_lookup(xt, feature_matrix, b, s, d)
    return out5.transpose(2, 4, 0, 1, 3).reshape(b, s, d)


# speedup vs baseline: 10.0399x; 1.0047x over previous
import functools

import jax
import jax.numpy as jnp
from jax import lax
from jax.experimental import pallas as pl
from jax.experimental.pallas import tpu as pltpu
from jax.experimental.pallas import tpu_sc as plsc

_NC = 2
_NS = 16
_NW = _NC * _NS
_L = 16
_BT = 128


@functools.partial(jax.jit, static_argnames=("b", "s", "d"))
def _emb_lookup(xt, feature_matrix, b, s, d):
    n_bt = b // _BT
    ft = d // 8
    st_n = s // 8
    assert n_bt == _NW and d % 8 == 0
    v = feature_matrix.shape[0]
    mesh = plsc.VectorSubcoreMesh(core_axis_name="c", subcore_axis_name="s")

    @functools.partial(
        pl.kernel,
        mesh=mesh,
        compiler_params=pltpu.CompilerParams(
            use_tc_tiling_on_sc=False, needs_layout_passes=False
        ),
        out_type=jax.ShapeDtypeStruct((s, ft, n_bt, 8, _BT), jnp.float32),
        scratch_types=[
            pltpu.VMEM((v, d), jnp.float32),
            pltpu.VMEM((st_n, 8, _BT), jnp.int32),
            pltpu.VMEM((4, ft, 8, _BT), jnp.float32),
            pltpu.SemaphoreType.DMA,
            pltpu.SemaphoreType.DMA,
            pltpu.SemaphoreType.DMA,
            pltpu.SemaphoreType.DMA,
        ],
    )
    def emb(x_hbm, tab_hbm, out_hbm, tab_v, idx_v, tile_v,
            osem0, osem1, osem2, osem3):
        wid = lax.axis_index("s") * _NC + lax.axis_index("c")
        pltpu.sync_copy(tab_hbm, tab_v)
        pltpu.sync_copy(x_hbm.at[:, wid], idx_v)

        f_vecs = [jnp.full((_L,), f, jnp.int32) for f in range(d)]
        osems = (osem0, osem1, osem2, osem3)
        depth = 4

        def unit(u, slot):
            @pl.when(u >= depth)
            def _():
                pltpu.make_async_copy(
                    tile_v.at[slot],
                    out_hbm.at[0, pl.ds(0, ft), wid],
                    osems[slot],
                ).wait()

            st = u // 8
            si = lax.rem(u, 8)
            idx_gs = [
                idx_v[st, si, pl.ds(g * _L, _L)] for g in range(_BT // _L)
            ]
            pairs = [(f, g) for g in range(_BT // _L) for f in range(d)]
            lag = 8
            pend = {}
            for i, (f, g) in enumerate(pairs):
                pend[i] = plsc.load_gather(tab_v, [idx_gs[g], f_vecs[f]])
                if i >= lag:
                    pf, pg = pairs[i - lag]
                    tile_v[slot, pf // 8, pf % 8, pl.ds(pg * _L, _L)] = (
                        pend.pop(i - lag)
                    )
            for i in range(len(pairs) - lag, len(pairs)):
                pf, pg = pairs[i]
                tile_v[slot, pf // 8, pf % 8, pl.ds(pg * _L, _L)] = (
                    pend.pop(i)
                )

            pltpu.async_copy(
                tile_v.at[slot],
                out_hbm.at[u, pl.ds(0, ft), wid],
                osems[slot],
            )

        def body(t, carry):
            for k in range(depth):
                unit(depth * t + k, k)
            return carry

        lax.fori_loop(0, s // depth, body, 0)
        for slot in range(depth):
            pltpu.make_async_copy(
                tile_v.at[slot],
                out_hbm.at[0, pl.ds(0, ft), wid],
                osems[slot],
            ).wait()

    return emb(xt, feature_matrix)


def kernel(x, feature_matrix):
    b, s = x.shape
    v, d = feature_matrix.shape
    assert b % _BT == 0 and b // _BT == _NW and s % 8 == 0
    xt = (
        x.astype(jnp.int32)
        .T.reshape(s // 8, 8, _NW, _BT)
        .transpose(0, 2, 1, 3)
    )
    out5 = _emb_lookup(xt, feature_matrix, b, s, d)
    return out5.transpose(2, 4, 0, 1, 3).reshape(b, s, d)
